# baseline (speedup 1.0000x reference)
import functools

import numpy as np

import jax
import jax.numpy as jnp
from jax.experimental import pallas as pl
from jax.experimental.pallas import tpu as pltpu


def _dot(a, b):
    return jax.lax.dot_general(
        a, b, (((a.ndim - 1,), (0,)), ((), ())),
        preferred_element_type=jnp.float32,
        precision=jax.lax.Precision.DEFAULT,
    )


def _mega_kernel(il_ref, sl_ref, lol_ref, hil_ref, fl_ref, ll_ref,
                 adj_ref, x_ref, w1_ref, b1_ref, w2_ref, b2_ref, out_ref,
                 slab_ref, bufw_ref, buft_ref, sems_ref, semw_ref, semt_ref,
                 cat_ref, s2sc_ref, part_ref, acc_ref,
                 *, br, bc, n, nhid, nclass, w, tailw, e, nrow, nsteps):
    t = pl.program_id(0)

    def slab_copy(r):
        return pltpu.make_async_copy(
            adj_ref.at[pl.ds(pl.multiple_of(r * br, 8), br), :],
            slab_ref.at[jax.lax.rem(r, 2)],
            sems_ref.at[jax.lax.rem(r, 2)],
        )

    def wide_copy(u):
        row = pl.multiple_of(il_ref[u] * bc, 8)
        col = pl.multiple_of(sl_ref[u], 128)
        return pltpu.make_async_copy(
            adj_ref.at[pl.ds(row, bc), pl.ds(col, w)],
            bufw_ref.at[jax.lax.rem(u, 2)],
            semw_ref.at[jax.lax.rem(u, 2)],
        )

    def tail_copy(u):
        row = pl.multiple_of(il_ref[u] * bc, 8)
        slot = jax.lax.rem(il_ref[u], 2)
        return pltpu.make_async_copy(
            adj_ref.at[pl.ds(row, bc), pl.ds(e, tailw)],
            buft_ref.at[slot],
            semt_ref.at[slot],
        )

    @pl.when(t == 0)
    def _prologue():
        slab_copy(0).start()

    nxt = t + 1

    @pl.when(nxt < nrow)
    def _pre_slab():
        slab_copy(nxt).start()

    @pl.when((nxt >= nrow) & (nxt < nrow + nsteps))
    def _pre_wide():
        wide_copy(nxt - nrow).start()

    if tailw:
        @pl.when((nxt >= nrow) & (nxt < nrow + nsteps))
        def _pre_tail():
            u = nxt - nrow

            @pl.when(fl_ref[u] == 1)
            def _():
                tail_copy(u).start()

    @pl.when(t < nrow)
    def _phase1():
        r = t
        slab_copy(r).wait()

        @pl.when(r == 0)
        def _init():
            s2sc_ref[...] = jnp.zeros_like(s2sc_ref)
            cat_ref[:, nhid:] = jnp.zeros((n, nclass), jnp.float32)
            cat_ref[:, :nhid] = _dot(x_ref[...], w1_ref[...])

        c = ((r * br) // bc * bc) // 128 * 128

        @pl.when((jax.lax.rem(r, bc // br) == 0) & (r > 0))
        def _refresh():
            rows = jax.lax.broadcasted_iota(jnp.int32, (n, 1), 0)
            cat_ref[:, nhid:] = jnp.where(rows < c, s2sc_ref[...], 0.0)

        res = _dot(slab_ref[jax.lax.rem(r, 2)], cat_ref[...])
        part_ref[pl.ds(r * br, br), :] = res[:, nhid:]
        h = jnp.maximum(res[:, :nhid] + b1_ref[...], 0.0)
        s2sc_ref[pl.ds(r * br, br), :] = _dot(h, w2_ref[...])

    @pl.when(t >= nrow)
    def _phase2():
        u = t - nrow
        wide_copy(u).wait()

        @pl.when(fl_ref[u] == 1)
        def _zero():
            acc_ref[...] = jnp.zeros_like(acc_ref)

        s = pl.multiple_of(sl_ref[u], 128)
        lo = lol_ref[u]
        hi = hil_ref[u]
        g = jax.lax.broadcasted_iota(jnp.int32, (w, 1), 0) + s
        s2_blk = jnp.where((g >= lo) & (g < hi),
                           s2sc_ref[pl.ds(s, w), :], 0.0)
        acc_ref[...] += _dot(bufw_ref[jax.lax.rem(u, 2)], s2_blk)

        @pl.when(ll_ref[u] == 1)
        def _finish():
            i = il_ref[u]
            acc = acc_ref[...]
            if tailw:
                tail_copy(u).wait()
                acc = acc + _dot(buft_ref[jax.lax.rem(i, 2)],
                                 s2sc_ref[pl.ds(e, tailw), :])
            logits = acc + part_ref[pl.ds(i * bc, bc), :] + b2_ref[...]
            m = jnp.max(logits, axis=1, keepdims=True)
            lse = jnp.log(jnp.sum(jnp.exp(logits - m), axis=1, keepdims=True))
            out_ref[...] = logits - m - lse


def kernel(x, adj, W1, b1, W2, b2):
    n, nfeat = x.shape
    nhid = W1.shape[1]
    nclass = W2.shape[1]

    bc = min(1000, n)
    while n % bc or bc % 8:
        bc -= 1
    br = min(200, bc)
    while n % br or bc % br or br % 8:
        br -= 1
    nrow = n // br
    nblk = n // bc

    e = n // 128 * 128
    w = min(2048, e)
    tailw = n - e

    b1r = b1.reshape(1, nhid)
    b2r = b2.reshape(1, nclass)

    il, sl, lol, hil, fl, ll = [], [], [], [], [], []
    for i in range(nblk):
        m_i = (i * bc) // 128 * 128
        nk = max(1, -(-(e - m_i) // w))
        for k in range(nk):
            cov_lo = m_i + k * w
            cov_hi = min(cov_lo + w, e)
            start = min(cov_lo, e - w)
            il.append(i)
            sl.append(start)
            lol.append(cov_lo)
            hil.append(cov_hi)
            fl.append(1 if k == 0 else 0)
            ll.append(1 if k == nk - 1 else 0)
    nsteps = len(il)
    lists = [jnp.asarray(np.array(v + [v[-1]], dtype=np.int32))
             for v in (il, sl, lol, hil, fl, ll)]

    grid_spec = pltpu.PrefetchScalarGridSpec(
        num_scalar_prefetch=6,
        grid=(nrow + nsteps,),
        in_specs=[
            pl.BlockSpec(memory_space=pltpu.MemorySpace.HBM),
            pl.BlockSpec((n, nfeat), lambda t, *pf: (0, 0)),
            pl.BlockSpec((nfeat, nhid), lambda t, *pf: (0, 0)),
            pl.BlockSpec((1, nhid), lambda t, *pf: (0, 0)),
            pl.BlockSpec((nhid, nclass), lambda t, *pf: (0, 0)),
            pl.BlockSpec((1, nclass), lambda t, *pf: (0, 0)),
        ],
        out_specs=pl.BlockSpec(
            (bc, nclass),
            lambda t, *pf: (pf[0][jnp.maximum(t - nrow, 0)], 0)),
        scratch_shapes=[
            pltpu.VMEM((2, br, n), jnp.float32),
            pltpu.VMEM((2, bc, w), jnp.float32),
            pltpu.VMEM((2, bc, max(tailw, 1)), jnp.float32),
            pltpu.SemaphoreType.DMA((2,)),
            pltpu.SemaphoreType.DMA((2,)),
            pltpu.SemaphoreType.DMA((2,)),
            pltpu.VMEM((n, nhid + nclass), jnp.float32),
            pltpu.VMEM((n, nclass), jnp.float32),
            pltpu.VMEM((n, nclass), jnp.float32),
            pltpu.VMEM((bc, nclass), jnp.float32),
        ],
    )

    out = pl.pallas_call(
        functools.partial(_mega_kernel, br=br, bc=bc, n=n, nhid=nhid,
                          nclass=nclass, w=w, tailw=tailw, e=e, nrow=nrow,
                          nsteps=nsteps),
        grid_spec=grid_spec,
        out_shape=jax.ShapeDtypeStruct((n, nclass), jnp.float32),
        compiler_params=pltpu.CompilerParams(
            dimension_semantics=("arbitrary",),
        ),
    )(*lists, adj, x, W1, b1r, W2, b2r)

    return out


# speedup vs baseline: 1.0255x; 1.0255x over previous
import functools

import numpy as np

import jax
import jax.numpy as jnp
from jax.experimental import pallas as pl
from jax.experimental.pallas import tpu as pltpu


def _dot(a, b):
    return jax.lax.dot_general(
        a, b, (((a.ndim - 1,), (0,)), ((), ())),
        preferred_element_type=jnp.float32,
        precision=jax.lax.Precision.DEFAULT,
    )


def _mega_kernel(il_ref, sl_ref, lol_ref, hil_ref, fl_ref, ll_ref,
                 adj_ref, x_ref, w1_ref, b1_ref, w2_ref, b2_ref, out_ref,
                 slab_ref, bufw_ref, buft_ref, sems_ref, semw_ref, semt_ref,
                 cat_ref, s2sc_ref, part_ref, acc_ref,
                 *, br, bc, n, nhid, nclass, w, tailw, e, nrow, nsteps):
    t = pl.program_id(0)

    def slab_copy(r):
        return pltpu.make_async_copy(
            adj_ref.at[pl.ds(pl.multiple_of(r * br, 8), br), :],
            slab_ref.at[jax.lax.rem(r, 3)],
            sems_ref.at[jax.lax.rem(r, 3)],
        )

    def wide_copy(u):
        row = pl.multiple_of(il_ref[u] * bc, 8)
        col = pl.multiple_of(sl_ref[u], 128)
        return pltpu.make_async_copy(
            adj_ref.at[pl.ds(row, bc), pl.ds(col, w)],
            bufw_ref.at[jax.lax.rem(u, 2)],
            semw_ref.at[jax.lax.rem(u, 2)],
        )

    def tail_copy(u):
        row = pl.multiple_of(il_ref[u] * bc, 8)
        slot = jax.lax.rem(il_ref[u], 2)
        return pltpu.make_async_copy(
            adj_ref.at[pl.ds(row, bc), pl.ds(e, tailw)],
            buft_ref.at[slot],
            semt_ref.at[slot],
        )

    @pl.when(t == 0)
    def _prologue():
        slab_copy(0).start()
        if nrow > 1:
            slab_copy(1).start()

    nxt = t + 1

    @pl.when(t + 2 < nrow)
    def _pre_slab():
        slab_copy(t + 2).start()

    @pl.when((nxt >= nrow) & (nxt < nrow + nsteps))
    def _pre_wide():
        wide_copy(nxt - nrow).start()

    if tailw:
        @pl.when((nxt >= nrow) & (nxt < nrow + nsteps))
        def _pre_tail():
            u = nxt - nrow

            @pl.when(fl_ref[u] == 1)
            def _():
                tail_copy(u).start()

    @pl.when(t < nrow)
    def _phase1():
        r = t
        slab_copy(r).wait()

        @pl.when(r == 0)
        def _init():
            s2sc_ref[...] = jnp.zeros_like(s2sc_ref)
            cat_ref[:, nhid:] = jnp.zeros((n, nclass), jnp.float32)
            cat_ref[:, :nhid] = _dot(x_ref[...], w1_ref[...])

        c = ((r * br) // bc * bc) // 128 * 128

        @pl.when((jax.lax.rem(r, bc // br) == 0) & (r > 0))
        def _refresh():
            rows = jax.lax.broadcasted_iota(jnp.int32, (n, 1), 0)
            cat_ref[:, nhid:] = jnp.where(rows < c, s2sc_ref[...], 0.0)

        res = _dot(slab_ref[jax.lax.rem(r, 3)], cat_ref[...])
        part_ref[pl.ds(r * br, br), :] = res[:, nhid:]
        h = jnp.maximum(res[:, :nhid] + b1_ref[...], 0.0)
        s2sc_ref[pl.ds(r * br, br), :] = _dot(h, w2_ref[...])

    @pl.when(t >= nrow)
    def _phase2():
        u = t - nrow
        wide_copy(u).wait()

        @pl.when(fl_ref[u] == 1)
        def _zero():
            acc_ref[...] = jnp.zeros_like(acc_ref)

        s = pl.multiple_of(sl_ref[u], 128)
        lo = lol_ref[u]
        hi = hil_ref[u]
        g = jax.lax.broadcasted_iota(jnp.int32, (w, 1), 0) + s
        s2_blk = jnp.where((g >= lo) & (g < hi),
                           s2sc_ref[pl.ds(s, w), :], 0.0)
        acc_ref[...] += _dot(bufw_ref[jax.lax.rem(u, 2)], s2_blk)

        @pl.when(ll_ref[u] == 1)
        def _finish():
            i = il_ref[u]
            acc = acc_ref[...]
            if tailw:
                tail_copy(u).wait()
                acc = acc + _dot(buft_ref[jax.lax.rem(i, 2)],
                                 s2sc_ref[pl.ds(e, tailw), :])
            logits = acc + part_ref[pl.ds(i * bc, bc), :] + b2_ref[...]
            m = jnp.max(logits, axis=1, keepdims=True)
            lse = jnp.log(jnp.sum(jnp.exp(logits - m), axis=1, keepdims=True))
            out_ref[...] = logits - m - lse


def kernel(x, adj, W1, b1, W2, b2):
    n, nfeat = x.shape
    nhid = W1.shape[1]
    nclass = W2.shape[1]

    bc = min(1000, n)
    while n % bc or bc % 8:
        bc -= 1
    br = min(200, bc)
    while n % br or bc % br or br % 8:
        br -= 1
    nrow = n // br
    nblk = n // bc

    e = n // 128 * 128
    w = min(1024, e)
    tailw = n - e

    b1r = b1.reshape(1, nhid)
    b2r = b2.reshape(1, nclass)

    il, sl, lol, hil, fl, ll = [], [], [], [], [], []
    for i in range(nblk):
        m_i = (i * bc) // 128 * 128
        nk = max(1, -(-(e - m_i) // w))
        for k in range(nk):
            cov_lo = m_i + k * w
            cov_hi = min(cov_lo + w, e)
            start = min(cov_lo, e - w)
            il.append(i)
            sl.append(start)
            lol.append(cov_lo)
            hil.append(cov_hi)
            fl.append(1 if k == 0 else 0)
            ll.append(1 if k == nk - 1 else 0)
    nsteps = len(il)
    lists = [jnp.asarray(np.array(v + [v[-1]], dtype=np.int32))
             for v in (il, sl, lol, hil, fl, ll)]

    grid_spec = pltpu.PrefetchScalarGridSpec(
        num_scalar_prefetch=6,
        grid=(nrow + nsteps,),
        in_specs=[
            pl.BlockSpec(memory_space=pltpu.MemorySpace.HBM),
            pl.BlockSpec((n, nfeat), lambda t, *pf: (0, 0)),
            pl.BlockSpec((nfeat, nhid), lambda t, *pf: (0, 0)),
            pl.BlockSpec((1, nhid), lambda t, *pf: (0, 0)),
            pl.BlockSpec((nhid, nclass), lambda t, *pf: (0, 0)),
            pl.BlockSpec((1, nclass), lambda t, *pf: (0, 0)),
        ],
        out_specs=pl.BlockSpec(
            (bc, nclass),
            lambda t, *pf: (pf[0][jnp.maximum(t - nrow, 0)], 0)),
        scratch_shapes=[
            pltpu.VMEM((3, br, n), jnp.float32),
            pltpu.VMEM((2, bc, w), jnp.float32),
            pltpu.VMEM((2, bc, max(tailw, 1)), jnp.float32),
            pltpu.SemaphoreType.DMA((3,)),
            pltpu.SemaphoreType.DMA((2,)),
            pltpu.SemaphoreType.DMA((2,)),
            pltpu.VMEM((n, nhid + nclass), jnp.float32),
            pltpu.VMEM((n, nclass), jnp.float32),
            pltpu.VMEM((n, nclass), jnp.float32),
            pltpu.VMEM((bc, nclass), jnp.float32),
        ],
    )

    out = pl.pallas_call(
        functools.partial(_mega_kernel, br=br, bc=bc, n=n, nhid=nhid,
                          nclass=nclass, w=w, tailw=tailw, e=e, nrow=nrow,
                          nsteps=nsteps),
        grid_spec=grid_spec,
        out_shape=jax.ShapeDtypeStruct((n, nclass), jnp.float32),
        compiler_params=pltpu.CompilerParams(
            dimension_semantics=("arbitrary",),
        ),
    )(*lists, adj, x, W1, b1r, W2, b2r)

    return out


# speedup vs baseline: 1.1424x; 1.1140x over previous
import functools

import numpy as np

import jax
import jax.numpy as jnp
from jax.experimental import pallas as pl
from jax.experimental.pallas import tpu as pltpu


def _dot(a, b):
    return jax.lax.dot_general(
        a, b, (((a.ndim - 1,), (0,)), ((), ())),
        preferred_element_type=jnp.float32,
        precision=jax.lax.Precision.DEFAULT,
    )


def _mega_kernel(il_ref, sl_ref, lol_ref, hil_ref, fl_ref, ll_ref,
                 adj_ref, x_ref, w1_ref, b1_ref, w2_ref, b2_ref, out_ref,
                 slab_ref, bufw_ref, buft_ref, sems_ref, semw_ref, semt_ref,
                 cat_ref, s2sc_ref, part_ref, acc_ref,
                 *, br, bc, n, nhid, nclass, w, tailw, e, nrow, nsteps):
    t = pl.program_id(0)

    def slab_copy(r):
        return pltpu.make_async_copy(
            adj_ref.at[pl.ds(pl.multiple_of(r * br, 8), br), :],
            slab_ref.at[jax.lax.rem(r, 3)],
            sems_ref.at[jax.lax.rem(r, 3)],
        )

    def wide_copy(u):
        row = pl.multiple_of(il_ref[u] * bc, 8)
        col = pl.multiple_of(sl_ref[u], 128)
        return pltpu.make_async_copy(
            adj_ref.at[pl.ds(row, bc), pl.ds(col, w)],
            bufw_ref.at[jax.lax.rem(u, 3)],
            semw_ref.at[jax.lax.rem(u, 3)],
        )

    def tail_copy(u):
        row = pl.multiple_of(il_ref[u] * bc, 8)
        slot = jax.lax.rem(il_ref[u], 2)
        return pltpu.make_async_copy(
            adj_ref.at[pl.ds(row, bc), pl.ds(e, tailw)],
            buft_ref.at[slot],
            semt_ref.at[slot],
        )

    @pl.when(t == 0)
    def _prologue():
        slab_copy(0).start()
        if nrow > 1:
            slab_copy(1).start()

    nxt = t + 1

    @pl.when(t + 2 < nrow)
    def _pre_slab():
        slab_copy(t + 2).start()

    nxt2 = t + 2

    @pl.when((nxt2 >= nrow) & (nxt2 < nrow + nsteps))
    def _pre_wide():
        wide_copy(nxt2 - nrow).start()

    if tailw:
        @pl.when((nxt2 >= nrow) & (nxt2 < nrow + nsteps))
        def _pre_tail():
            u = nxt2 - nrow

            @pl.when(fl_ref[u] == 1)
            def _():
                tail_copy(u).start()

    @pl.when(t < nrow)
    def _phase1():
        r = t

        @pl.when(r == 0)
        def _init():
            s2sc_ref[...] = jnp.zeros_like(s2sc_ref)
            cat_ref[:, nhid:] = jnp.zeros((n, nclass), jnp.float32)
            cat_ref[:, :nhid] = _dot(x_ref[...], w1_ref[...])

        c = ((r * br) // bc * bc) // 128 * 128

        @pl.when((jax.lax.rem(r, bc // br) == 0) & (r > 0))
        def _refresh():
            wr = min(bc // 128 * 128 + 128, n)
            base = pl.multiple_of(jnp.maximum(c - wr, 0), 8)
            rows = jax.lax.broadcasted_iota(jnp.int32, (wr, 1), 0) + base
            cat_ref[pl.ds(base, wr), nhid:] = jnp.where(
                rows < c, s2sc_ref[pl.ds(base, wr), :], 0.0)

        slab_copy(r).wait()
        res = _dot(slab_ref[jax.lax.rem(r, 3)], cat_ref[...])
        part_ref[pl.ds(r * br, br), :] = res[:, nhid:]
        h = jnp.maximum(res[:, :nhid] + b1_ref[...], 0.0)
        s2sc_ref[pl.ds(r * br, br), :] = _dot(h, w2_ref[...])

    @pl.when(t >= nrow)
    def _phase2():
        u = t - nrow

        @pl.when(fl_ref[u] == 1)
        def _zero():
            acc_ref[...] = jnp.zeros_like(acc_ref)

        s = pl.multiple_of(sl_ref[u], 128)
        lo = lol_ref[u]
        hi = hil_ref[u]
        g = jax.lax.broadcasted_iota(jnp.int32, (w, 1), 0) + s
        s2_blk = jnp.where((g >= lo) & (g < hi),
                           s2sc_ref[pl.ds(s, w), :], 0.0)
        wide_copy(u).wait()
        acc_ref[...] += _dot(bufw_ref[jax.lax.rem(u, 3)], s2_blk)

        @pl.when(ll_ref[u] == 1)
        def _finish():
            i = il_ref[u]
            acc = acc_ref[...]
            if tailw:
                tail_copy(u).wait()
                acc = acc + _dot(buft_ref[jax.lax.rem(i, 2)],
                                 s2sc_ref[pl.ds(e, tailw), :])
            logits = acc + part_ref[pl.ds(i * bc, bc), :] + b2_ref[...]
            m = jnp.max(logits, axis=1, keepdims=True)
            lse = jnp.log(jnp.sum(jnp.exp(logits - m), axis=1, keepdims=True))
            out_ref[...] = logits - m - lse


def kernel(x, adj, W1, b1, W2, b2):
    n, nfeat = x.shape
    nhid = W1.shape[1]
    nclass = W2.shape[1]

    bc = min(1000, n)
    while n % bc or bc % 8:
        bc -= 1
    br = min(200, bc)
    while n % br or bc % br or br % 8:
        br -= 1
    nrow = n // br
    nblk = n // bc

    e = n // 128 * 128
    w = min(1024, e)
    tailw = n - e

    b1r = b1.reshape(1, nhid)
    b2r = b2.reshape(1, nclass)

    il, sl, lol, hil, fl, ll = [], [], [], [], [], []
    for i in range(nblk):
        m_i = (i * bc) // 128 * 128
        nk = max(1, -(-(e - m_i) // w))
        for k in range(nk):
            cov_lo = m_i + k * w
            cov_hi = min(cov_lo + w, e)
            start = min(cov_lo, e - w)
            il.append(i)
            sl.append(start)
            lol.append(cov_lo)
            hil.append(cov_hi)
            fl.append(1 if k == 0 else 0)
            ll.append(1 if k == nk - 1 else 0)
    nsteps = len(il)
    lists = [jnp.asarray(np.array(v + [v[-1]], dtype=np.int32))
             for v in (il, sl, lol, hil, fl, ll)]

    grid_spec = pltpu.PrefetchScalarGridSpec(
        num_scalar_prefetch=6,
        grid=(nrow + nsteps,),
        in_specs=[
            pl.BlockSpec(memory_space=pltpu.MemorySpace.HBM),
            pl.BlockSpec((n, nfeat), lambda t, *pf: (0, 0)),
            pl.BlockSpec((nfeat, nhid), lambda t, *pf: (0, 0)),
            pl.BlockSpec((1, nhid), lambda t, *pf: (0, 0)),
            pl.BlockSpec((nhid, nclass), lambda t, *pf: (0, 0)),
            pl.BlockSpec((1, nclass), lambda t, *pf: (0, 0)),
        ],
        out_specs=pl.BlockSpec(
            (bc, nclass),
            lambda t, *pf: (pf[0][jnp.maximum(t - nrow, 0)], 0)),
        scratch_shapes=[
            pltpu.VMEM((3, br, n), jnp.float32),
            pltpu.VMEM((3, bc, w), jnp.float32),
            pltpu.VMEM((2, bc, max(tailw, 1)), jnp.float32),
            pltpu.SemaphoreType.DMA((3,)),
            pltpu.SemaphoreType.DMA((3,)),
            pltpu.SemaphoreType.DMA((2,)),
            pltpu.VMEM((n, nhid + nclass), jnp.float32),
            pltpu.VMEM((n, nclass), jnp.float32),
            pltpu.VMEM((n, nclass), jnp.float32),
            pltpu.VMEM((bc, nclass), jnp.float32),
        ],
    )

    out = pl.pallas_call(
        functools.partial(_mega_kernel, br=br, bc=bc, n=n, nhid=nhid,
                          nclass=nclass, w=w, tailw=tailw, e=e, nrow=nrow,
                          nsteps=nsteps),
        grid_spec=grid_spec,
        out_shape=jax.ShapeDtypeStruct((n, nclass), jnp.float32),
        compiler_params=pltpu.CompilerParams(
            dimension_semantics=("arbitrary",),
        ),
    )(*lists, adj, x, W1, b1r, W2, b2r)

    return out
